# baseline (speedup 1.0000x reference)
import jax
import jax.numpy as jnp
import numpy as np
from jax.experimental import pallas as pl
from jax.experimental.pallas import tpu as pltpu

_N = 20000
_NP = 20480
_NQB = _NP // 512
_PRE = 6000
_NPAD = 6144
_BLK = 512
_NBLK = _NPAD // _BLK
_COUNT = 1000
_OUTPAD = 1024
_THR = 0.7
_WIN = 768
_LOBASE = 1000000


def _transform_rows(a):
    sc = a[0:1]
    ay1, ax1, ay2, ax2 = a[1:2], a[2:3], a[3:4], a[4:5]
    d0, d1 = a[5:6] * 0.1, a[6:7] * 0.1
    d2, d3 = a[7:8] * 0.2, a[8:9] * 0.2
    h = ay2 - ay1
    w = ax2 - ax1
    cy = ay1 + 0.5 * h + d0 * h
    cx = ax1 + 0.5 * w + d1 * w
    h = h * jnp.exp(d2)
    w = w * jnp.exp(d3)
    y1 = cy - 0.5 * h
    x1 = cx - 0.5 * w
    y2 = jnp.clip(y1 + h, 0.0, 1.0)
    x2 = jnp.clip(x1 + w, 0.0, 1.0)
    y1 = jnp.clip(y1, 0.0, 1.0)
    x1 = jnp.clip(x1, 0.0, 1.0)
    z = jnp.zeros_like(sc)
    return jnp.concatenate([y1, x1, y2, x2, sc, z, z, z], axis=0)


def _transform_cols(c):
    sc = c[:, 0:1]
    ay1, ax1, ay2, ax2 = c[:, 1:2], c[:, 2:3], c[:, 3:4], c[:, 4:5]
    d0, d1 = c[:, 5:6] * 0.1, c[:, 6:7] * 0.1
    d2, d3 = c[:, 7:8] * 0.2, c[:, 8:9] * 0.2
    h = ay2 - ay1
    w = ax2 - ax1
    cy = ay1 + 0.5 * h + d0 * h
    cx = ax1 + 0.5 * w + d1 * w
    h = h * jnp.exp(d2)
    w = w * jnp.exp(d3)
    y1 = cy - 0.5 * h
    x1 = cx - 0.5 * w
    y2 = jnp.clip(y1 + h, 0.0, 1.0)
    x2 = jnp.clip(x1 + w, 0.0, 1.0)
    y1 = jnp.clip(y1, 0.0, 1.0)
    x1 = jnp.clip(x1, 0.0, 1.0)
    z = jnp.zeros_like(sc)
    return jnp.concatenate([y1, x1, y2, x2, sc, z, z, z], axis=1)


def _iou_thr(y1p, x1p, y2p, x2p, area_p, y1c, x1c, y2c, x2c, area_c):
    yy1 = jnp.maximum(y1p, y1c)
    xx1 = jnp.maximum(x1p, x1c)
    yy2 = jnp.minimum(y2p, y2c)
    xx2 = jnp.minimum(x2p, x2c)
    inter = jnp.maximum(yy2 - yy1, 0.0) * jnp.maximum(xx2 - xx1, 0.0)
    iou = inter / (area_p + area_c - inter + 1e-8)
    return iou > _THR


def _body(rows_ref, cols_ref, out_ref,
          ccols_ref, crows_ref, scols_ref, srows_ref,
          rank_ref, rcol_ref, mcol_ref, keep_ref):
    f32 = jnp.float32

    hi_full = jax.lax.bitcast_convert_type(rows_ref[0, 0:1, :], jnp.int32)

    def cnt_hi(m):
        return jnp.sum((hi_full > m).astype(f32))

    def bis1(_, st):
        lob, hib = st
        mid = lob + (hib - lob) // 2
        big = cnt_hi(mid) >= float(_PRE)
        return jnp.where(big, mid, lob), jnp.where(big, hib, mid)

    lob0 = jnp.int32(-1)
    hib0 = jnp.int32(1 << 30)
    _, hstar = jax.lax.fori_loop(0, 31, bis1, (lob0, hib0))
    t_need = float(_PRE) - cnt_hi(hstar)

    lo_full = _LOBASE - jax.lax.broadcasted_iota(jnp.int32, (1, _NP), 1)
    tie_full = (hi_full == hstar).astype(f32)

    def cnt_lo(l):
        return jnp.sum(tie_full * (lo_full > l).astype(f32))

    def bis2(_, st):
        lob, hib = st
        mid = lob + (hib - lob) // 2
        big = cnt_lo(mid) >= t_need
        return jnp.where(big, mid, lob), jnp.where(big, hib, mid)

    _, lstar = jax.lax.fori_loop(
        0, 15, bis2, (jnp.int32(_LOBASE - _NP - 1), jnp.int32(_LOBASE + 1)))

    pio = jax.lax.broadcasted_iota(jnp.int32, (_BLK, _BLK), 0)
    cio = jax.lax.broadcasted_iota(jnp.int32, (_BLK, _BLK), 1)
    ut = jnp.where(pio <= cio, 1.0, 0.0).astype(f32)
    lt = jnp.where(pio >= cio, 1.0, 0.0).astype(f32)
    oio = jax.lax.broadcasted_iota(jnp.int32, (_OUTPAD, _BLK), 0)
    wio = jax.lax.broadcasted_iota(jnp.int32, (_WIN, _BLK), 0)
    wio_t = jax.lax.broadcasted_iota(jnp.int32, (_BLK, _WIN), 1)

    ccols_ref[...] = jnp.zeros_like(ccols_ref)
    crows_ref[...] = jnp.zeros_like(crows_ref)

    def compact_body(qb, R):
        base = qb * _BLK
        ra = pl.multiple_of((R // 128) * 128, 128)
        d = R - ra

        a_blk = rows_ref[0, :, pl.ds(base, _BLK)]
        vr = _transform_rows(a_blk)
        hi_r = jax.lax.bitcast_convert_type(a_blk[0:1], jnp.int32)
        lo_r = _LOBASE - (base + jax.lax.broadcasted_iota(jnp.int32, (1, _BLK), 1))
        sel_r = (hi_r > hstar) | ((hi_r == hstar) & (lo_r >= lstar))
        m_r = sel_r.astype(f32)
        cum_r = jnp.dot(m_r, ut, preferred_element_type=f32)
        ploc_r = (cum_r - 1.0).astype(jnp.int32) + d
        p_b = jnp.where(sel_r & (ploc_r == wio), 1.0, 0.0).astype(f32)

        c_blk = cols_ref[0, pl.ds(base, _BLK), :]
        vc = _transform_cols(c_blk)
        hi_c = jax.lax.bitcast_convert_type(c_blk[:, 0:1], jnp.int32)
        lo_c = _LOBASE - (base + jax.lax.broadcasted_iota(jnp.int32, (_BLK, 1), 0))
        sel_c = (hi_c > hstar) | ((hi_c == hstar) & (lo_c >= lstar))
        m_c = sel_c.astype(f32)
        cum_c = jnp.dot(lt, m_c, preferred_element_type=f32)
        ploc_c = (cum_c - 1.0).astype(jnp.int32) + d
        p_bt = jnp.where(sel_c & (ploc_c == wio_t), 1.0, 0.0).astype(f32)

        ccols_ref[pl.ds(ra, _WIN), :] = ccols_ref[pl.ds(ra, _WIN), :] + jnp.dot(
            p_b, vc, preferred_element_type=f32,
            precision=jax.lax.Precision.HIGHEST)
        crows_ref[:, pl.ds(ra, _WIN)] = crows_ref[:, pl.ds(ra, _WIN)] + jnp.dot(
            vr, p_bt, preferred_element_type=f32,
            precision=jax.lax.Precision.HIGHEST)
        return R + jnp.sum(m_r).astype(jnp.int32)

    jax.lax.fori_loop(0, _NQB, compact_body, jnp.int32(0))

    ones_row = jnp.ones((1, _BLK), f32)
    ones_col = jnp.ones((_BLK, 1), f32)
    rcol_ref[...] = jnp.zeros_like(rcol_ref)

    def rank_outer(cb, _):
        s_c = crows_ref[4:5, pl.ds(cb * _BLK, _BLK)]

        def rank_inner(pb, acc):
            s_p = ccols_ref[pl.ds(pb * _BLK, _BLK), 4:5]
            tie_lt = (pb * _BLK + pio) < (cb * _BLK + cio)
            cmp = jnp.where(
                (s_p > s_c) | ((s_p == s_c) & tie_lt), 1.0, 0.0).astype(f32)
            po = pl.ds(pb * _BLK, _BLK)
            rcol_ref[po, 0:1] = rcol_ref[po, 0:1] + jnp.dot(
                cmp, ones_col, preferred_element_type=f32)
            return acc + jnp.dot(ones_row, cmp, preferred_element_type=f32)

        acc = jax.lax.fori_loop(0, _NBLK, rank_inner, jnp.zeros((1, _BLK), f32))
        rank_ref[0:1, pl.ds(cb * _BLK, _BLK)] = acc
        return _

    jax.lax.fori_loop(0, _NBLK, rank_outer, jnp.int32(0))

    def perm_outer(rb, _):
        def perm_inner(ib, acc):
            acc_c, acc_r = acc
            rk_row = rank_ref[0:1, pl.ds(ib * _BLK, _BLK)].astype(jnp.int32)
            p2 = jnp.where(rk_row - rb * _BLK == pio, 1.0, 0.0).astype(f32)
            acc_c = acc_c + jnp.dot(
                p2, ccols_ref[pl.ds(ib * _BLK, _BLK), :],
                preferred_element_type=f32, precision=jax.lax.Precision.HIGHEST)
            rk_col = (float(_NPAD - 1)
                      - rcol_ref[pl.ds(ib * _BLK, _BLK), 0:1]).astype(jnp.int32)
            p2t = jnp.where(rk_col - rb * _BLK == cio, 1.0, 0.0).astype(f32)
            acc_r = acc_r + jnp.dot(
                crows_ref[:, pl.ds(ib * _BLK, _BLK)], p2t,
                preferred_element_type=f32, precision=jax.lax.Precision.HIGHEST)
            return acc_c, acc_r

        acc_c, acc_r = jax.lax.fori_loop(
            0, _NBLK, perm_inner,
            (jnp.zeros((_BLK, 8), f32), jnp.zeros((8, _BLK), f32)))
        scols_ref[pl.ds(rb * _BLK, _BLK), :] = acc_c
        srows_ref[:, pl.ds(rb * _BLK, _BLK)] = acc_r
        return _

    jax.lax.fori_loop(0, _NBLK, perm_outer, jnp.int32(0))

    srows = srows_ref[...]
    for i in range(_NBLK):
        c0 = i * _BLK
        y1c = srows[0:1, c0:c0 + _BLK]
        x1c = srows[1:2, c0:c0 + _BLK]
        y2c = srows[2:3, c0:c0 + _BLK]
        x2c = srows[3:4, c0:c0 + _BLK]
        area_c = jnp.maximum(y2c - y1c, 0.0) * jnp.maximum(x2c - x1c, 0.0)

        def cross_body(j, smax):
            pb = mcol_ref[pl.ds(j * _BLK, _BLK), :]
            y1p, x1p = pb[:, 0:1], pb[:, 1:2]
            y2p, x2p = pb[:, 2:3], pb[:, 3:4]
            area_p = jnp.maximum(y2p - y1p, 0.0) * jnp.maximum(x2p - x1p, 0.0)
            hitm = _iou_thr(y1p, x1p, y2p, x2p, area_p,
                            y1c, x1c, y2c, x2c, area_c)
            hit = jnp.max(jnp.where(hitm, 1.0, 0.0), axis=0, keepdims=True)
            return jnp.maximum(smax, hit)

        scross = jnp.zeros((1, _BLK), f32)
        if i > 0:
            scross = jax.lax.fori_loop(0, i, cross_body, scross)
        base = 1.0 - scross

        sb = scols_ref[c0:c0 + _BLK, 0:4]
        y1p, x1p = sb[:, 0:1], sb[:, 1:2]
        y2p, x2p = sb[:, 2:3], sb[:, 3:4]
        area_p = jnp.maximum(y2p - y1p, 0.0) * jnp.maximum(x2p - x1p, 0.0)
        m_sup = jnp.where(
            _iou_thr(y1p, x1p, y2p, x2p, area_p,
                     y1c, x1c, y2c, x2c, area_c) & (pio < cio),
            1.0, 0.0).astype(f32)

        def fix_cond(st):
            return st[1]

        def fix_body(st):
            alive, _ = st
            s = jnp.dot(alive, m_sup, preferred_element_type=f32)
            new_alive = jnp.where(s > 0.5, 0.0, base)
            return new_alive, jnp.any(new_alive != alive)

        s0 = jnp.dot(base, m_sup, preferred_element_type=f32)
        alive1 = jnp.where(s0 > 0.5, 0.0, base)
        alive, _ = jax.lax.while_loop(
            fix_cond, fix_body, (alive1, jnp.any(alive1 != base)))

        keep_ref[0:1, c0:c0 + _BLK] = alive
        ident = jnp.where(pio == cio, 1.0, 0.0).astype(f32)
        alive_col = jnp.sum(ident * alive, axis=1, keepdims=True)
        mcol_ref[c0:c0 + _BLK, :] = sb * alive_col

    def out_body(cb, run):
        kb = keep_ref[0:1, pl.ds(cb * _BLK, _BLK)]
        cum = jnp.dot(kb, ut, preferred_element_type=f32) + run
        kr = jnp.where(kb > 0.5, cum - 1.0, -5.0).astype(jnp.int32)
        p3 = jnp.where(kr == oio, 1.0, 0.0).astype(f32)
        src = scols_ref[pl.ds(cb * _BLK, _BLK), 0:4]
        out_ref[0] = out_ref[0] + jnp.dot(
            p3, src, preferred_element_type=f32,
            precision=jax.lax.Precision.HIGHEST)
        return run + jnp.sum(kb)

    out_ref[...] = jnp.zeros_like(out_ref)
    jax.lax.fori_loop(0, _NBLK, out_body, f32(0.0))


def _proposal_call(rows_all, cols_all, interpret=False):
    f32 = jnp.float32
    return pl.pallas_call(
        _body,
        grid=(rows_all.shape[0],),
        in_specs=[
            pl.BlockSpec((1, 16, _NP), lambda b: (b, 0, 0)),
            pl.BlockSpec((1, _NP, 16), lambda b: (b, 0, 0)),
        ],
        out_specs=pl.BlockSpec((1, _OUTPAD, 4), lambda b: (b, 0, 0)),
        out_shape=jax.ShapeDtypeStruct((rows_all.shape[0], _OUTPAD, 4), f32),
        scratch_shapes=[
            pltpu.VMEM((_NPAD + _OUTPAD, 8), f32),
            pltpu.VMEM((8, _NPAD + _OUTPAD), f32),
            pltpu.VMEM((_NPAD, 8), f32),
            pltpu.VMEM((8, _NPAD), f32),
            pltpu.VMEM((1, _NPAD), f32),
            pltpu.VMEM((_NPAD, 1), f32),
            pltpu.VMEM((_NPAD, 4), f32),
            pltpu.VMEM((1, _NPAD), f32),
        ],
        interpret=interpret,
    )(rows_all, cols_all)


def kernel(classes, bboxes, anchors):
    b = classes.shape[0]
    scores = classes[:, :, 1]
    sp = jnp.pad(scores, ((0, 0), (0, _NP - _N)), constant_values=-1.0)
    ap = jnp.pad(anchors, ((0, 0), (0, _NP - _N), (0, 0)))
    bp = jnp.pad(bboxes, ((0, 0), (0, _NP - _N), (0, 0)))
    cols_all = jnp.concatenate(
        [sp[..., None], ap, bp, jnp.zeros((b, _NP, 7), jnp.float32)], axis=2)
    rows_all = cols_all.transpose(0, 2, 1)
    out = _proposal_call(rows_all, cols_all)
    return out[:, :_COUNT, :]


# speedup vs baseline: 1.5967x; 1.5967x over previous
import jax
import jax.numpy as jnp
import numpy as np
from jax.experimental import pallas as pl
from jax.experimental.pallas import tpu as pltpu

_N = 20000
_NP = 20480
_NQB = _NP // 512
_PRE = 6000
_NPAD = 6144
_BLK = 512
_NBLK = _NPAD // _BLK
_COUNT = 1000
_OUTPAD = 1024
_THR = 0.7
_WIN = 768
_LOBASE = 1000000


def _transform_rows(a):
    sc = a[0:1]
    ay1, ax1, ay2, ax2 = a[1:2], a[2:3], a[3:4], a[4:5]
    d0, d1 = a[5:6] * 0.1, a[6:7] * 0.1
    d2, d3 = a[7:8] * 0.2, a[8:9] * 0.2
    h = ay2 - ay1
    w = ax2 - ax1
    cy = ay1 + 0.5 * h + d0 * h
    cx = ax1 + 0.5 * w + d1 * w
    h = h * jnp.exp(d2)
    w = w * jnp.exp(d3)
    y1 = cy - 0.5 * h
    x1 = cx - 0.5 * w
    y2 = jnp.clip(y1 + h, 0.0, 1.0)
    x2 = jnp.clip(x1 + w, 0.0, 1.0)
    y1 = jnp.clip(y1, 0.0, 1.0)
    x1 = jnp.clip(x1, 0.0, 1.0)
    z = jnp.zeros_like(sc)
    return jnp.concatenate([y1, x1, y2, x2, sc, z, z, z], axis=0)


def _transform_cols(c):
    sc = c[:, 0:1]
    ay1, ax1, ay2, ax2 = c[:, 1:2], c[:, 2:3], c[:, 3:4], c[:, 4:5]
    d0, d1 = c[:, 5:6] * 0.1, c[:, 6:7] * 0.1
    d2, d3 = c[:, 7:8] * 0.2, c[:, 8:9] * 0.2
    h = ay2 - ay1
    w = ax2 - ax1
    cy = ay1 + 0.5 * h + d0 * h
    cx = ax1 + 0.5 * w + d1 * w
    h = h * jnp.exp(d2)
    w = w * jnp.exp(d3)
    y1 = cy - 0.5 * h
    x1 = cx - 0.5 * w
    y2 = jnp.clip(y1 + h, 0.0, 1.0)
    x2 = jnp.clip(x1 + w, 0.0, 1.0)
    y1 = jnp.clip(y1, 0.0, 1.0)
    x1 = jnp.clip(x1, 0.0, 1.0)
    z = jnp.zeros_like(sc)
    return jnp.concatenate([y1, x1, y2, x2, sc, z, z, z], axis=1)


def _iou_thr(y1p, x1p, y2p, x2p, area_p, y1c, x1c, y2c, x2c, area_c):
    yy1 = jnp.maximum(y1p, y1c)
    xx1 = jnp.maximum(x1p, x1c)
    yy2 = jnp.minimum(y2p, y2c)
    xx2 = jnp.minimum(x2p, x2c)
    inter = jnp.maximum(yy2 - yy1, 0.0) * jnp.maximum(xx2 - xx1, 0.0)
    iou = inter / (area_p + area_c - inter + 1e-8)
    return iou > _THR


def _body(rows_ref, out_ref,
          crows_ref, scols_ref, srows_ref,
          rank_ref, rcol_ref, sccol_ref, mcol_ref, keep_ref):
    f32 = jnp.float32

    hi_full = jax.lax.bitcast_convert_type(rows_ref[0, 0:1, :], jnp.int32)

    def cnt_hi(m):
        return jnp.sum((hi_full > m).astype(f32))

    def bis1(_, st):
        lob, hib = st
        mid = lob + (hib - lob) // 2
        big = cnt_hi(mid) >= float(_PRE)
        return jnp.where(big, mid, lob), jnp.where(big, hib, mid)

    lob0 = jnp.int32(-1)
    hib0 = jnp.int32(1 << 30)
    _, hstar = jax.lax.fori_loop(0, 31, bis1, (lob0, hib0))
    t_need = float(_PRE) - cnt_hi(hstar)

    lo_full = _LOBASE - jax.lax.broadcasted_iota(jnp.int32, (1, _NP), 1)
    tie_full = (hi_full == hstar).astype(f32)

    def cnt_lo(l):
        return jnp.sum(tie_full * (lo_full > l).astype(f32))

    def bis2(_, st):
        lob, hib = st
        mid = lob + (hib - lob) // 2
        big = cnt_lo(mid) >= t_need
        return jnp.where(big, mid, lob), jnp.where(big, hib, mid)

    _, lstar = jax.lax.fori_loop(
        0, 15, bis2, (jnp.int32(_LOBASE - _NP - 1), jnp.int32(_LOBASE + 1)))

    pio = jax.lax.broadcasted_iota(jnp.int32, (_BLK, _BLK), 0)
    cio = jax.lax.broadcasted_iota(jnp.int32, (_BLK, _BLK), 1)
    ut = jnp.where(pio <= cio, 1.0, 0.0).astype(f32)
    lt = jnp.where(pio >= cio, 1.0, 0.0).astype(f32)
    oio = jax.lax.broadcasted_iota(jnp.int32, (_OUTPAD, _BLK), 0)
    wio = jax.lax.broadcasted_iota(jnp.int32, (_WIN, _BLK), 0)
    wio_t = jax.lax.broadcasted_iota(jnp.int32, (_BLK, _WIN), 1)

    crows_ref[...] = jnp.zeros_like(crows_ref)
    identi = jnp.where(pio == cio, 1, 0)

    def compact_body(qb, R):
        base = qb * _BLK
        ra = pl.multiple_of((R // 128) * 128, 128)
        d = R - ra

        a_blk = rows_ref[0, :, pl.ds(base, _BLK)]
        vr = _transform_rows(a_blk)
        hi_r = jax.lax.bitcast_convert_type(a_blk[0:1], jnp.int32)
        lo_r = _LOBASE - (base + jax.lax.broadcasted_iota(jnp.int32, (1, _BLK), 1))
        sel_r = (hi_r > hstar) | ((hi_r == hstar) & (lo_r >= lstar))
        m_r = sel_r.astype(f32)
        cum_r = jnp.dot(m_r, ut, preferred_element_type=f32)
        ploc_r = (cum_r - 1.0).astype(jnp.int32) + d
        enc_r = jnp.where(sel_r, ploc_r, -7)
        enc_c = jnp.sum(identi * enc_r, axis=1, keepdims=True)
        p_bt = jnp.where(enc_c == wio_t, 1.0, 0.0).astype(f32)

        crows_ref[:, pl.ds(ra, _WIN)] = crows_ref[:, pl.ds(ra, _WIN)] + jnp.dot(
            vr, p_bt, preferred_element_type=f32,
            precision=jax.lax.Precision.HIGHEST)
        return R + jnp.sum(m_r).astype(jnp.int32)

    jax.lax.fori_loop(0, _NQB, compact_body, jnp.int32(0))

    ones_row = jnp.ones((1, _BLK), f32)
    ones_col = jnp.ones((_BLK, 1), f32)
    rcol_ref[...] = jnp.zeros_like(rcol_ref)
    identf = jnp.where(pio == cio, 1.0, 0.0).astype(f32)

    def sccol_body(b2, _):
        row = crows_ref[4:5, pl.ds(b2 * _BLK, _BLK)]
        sccol_ref[pl.ds(b2 * _BLK, _BLK), 0:1] = jnp.sum(
            identf * row, axis=1, keepdims=True)
        return _

    jax.lax.fori_loop(0, _NBLK, sccol_body, jnp.int32(0))

    def rank_outer(cb, _):
        s_c = crows_ref[4:5, pl.ds(cb * _BLK, _BLK)]

        def rank_inner(pb, acc):
            s_p = sccol_ref[pl.ds(pb * _BLK, _BLK), 0:1]
            tie_lt = (pb * _BLK + pio) < (cb * _BLK + cio)
            cmp = jnp.where(
                (s_p > s_c) | ((s_p == s_c) & tie_lt), 1.0, 0.0).astype(f32)
            po = pl.ds(pb * _BLK, _BLK)
            rcol_ref[po, 0:1] = rcol_ref[po, 0:1] + jnp.dot(
                cmp, ones_col, preferred_element_type=f32)
            return acc + jnp.dot(ones_row, cmp, preferred_element_type=f32)

        acc = jax.lax.fori_loop(0, _NBLK, rank_inner, jnp.zeros((1, _BLK), f32))
        rank_ref[0:1, pl.ds(cb * _BLK, _BLK)] = acc
        return _

    jax.lax.fori_loop(0, _NBLK, rank_outer, jnp.int32(0))

    def perm_outer(rb, _):
        def perm_inner(ib, acc_r):
            rk_col = (float(_NPAD - 1)
                      - rcol_ref[pl.ds(ib * _BLK, _BLK), 0:1]).astype(jnp.int32)
            p2t = jnp.where(rk_col - rb * _BLK == cio, 1.0, 0.0).astype(f32)
            return acc_r + jnp.dot(
                crows_ref[:, pl.ds(ib * _BLK, _BLK)], p2t,
                preferred_element_type=f32, precision=jax.lax.Precision.HIGHEST)

        acc_r = jax.lax.fori_loop(
            0, _NBLK, perm_inner, jnp.zeros((8, _BLK), f32))
        srows_ref[:, pl.ds(rb * _BLK, _BLK)] = acc_r
        for c in range(4):
            scols_ref[pl.ds(rb * _BLK, _BLK), c:c + 1] = jnp.sum(
                identf * acc_r[c:c + 1, :], axis=1, keepdims=True)
        return _

    jax.lax.fori_loop(0, _NBLK, perm_outer, jnp.int32(0))

    srows = srows_ref[...]
    for i in range(_NBLK):
        c0 = i * _BLK
        y1c = srows[0:1, c0:c0 + _BLK]
        x1c = srows[1:2, c0:c0 + _BLK]
        y2c = srows[2:3, c0:c0 + _BLK]
        x2c = srows[3:4, c0:c0 + _BLK]
        area_c = jnp.maximum(y2c - y1c, 0.0) * jnp.maximum(x2c - x1c, 0.0)

        def cross_body(j, smax):
            pb = mcol_ref[pl.ds(j * _BLK, _BLK), :]
            y1p, x1p = pb[:, 0:1], pb[:, 1:2]
            y2p, x2p = pb[:, 2:3], pb[:, 3:4]
            area_p = jnp.maximum(y2p - y1p, 0.0) * jnp.maximum(x2p - x1p, 0.0)
            hitm = _iou_thr(y1p, x1p, y2p, x2p, area_p,
                            y1c, x1c, y2c, x2c, area_c)
            hit = jnp.max(jnp.where(hitm, 1.0, 0.0), axis=0, keepdims=True)
            return jnp.maximum(smax, hit)

        scross = jnp.zeros((1, _BLK), f32)
        if i > 0:
            scross = jax.lax.fori_loop(0, i, cross_body, scross)
        base = 1.0 - scross

        sb = scols_ref[c0:c0 + _BLK, :]
        y1p, x1p = sb[:, 0:1], sb[:, 1:2]
        y2p, x2p = sb[:, 2:3], sb[:, 3:4]
        area_p = jnp.maximum(y2p - y1p, 0.0) * jnp.maximum(x2p - x1p, 0.0)
        m_sup = jnp.where(
            _iou_thr(y1p, x1p, y2p, x2p, area_p,
                     y1c, x1c, y2c, x2c, area_c) & (pio < cio),
            1.0, 0.0).astype(f32)

        def fix_cond(st):
            return st[1]

        def fix_body(st):
            alive, _ = st
            s = jnp.dot(alive, m_sup, preferred_element_type=f32)
            new_alive = jnp.where(s > 0.5, 0.0, base)
            return new_alive, jnp.any(new_alive != alive)

        s0 = jnp.dot(base, m_sup, preferred_element_type=f32)
        alive1 = jnp.where(s0 > 0.5, 0.0, base)
        alive, _ = jax.lax.while_loop(
            fix_cond, fix_body, (alive1, jnp.any(alive1 != base)))

        keep_ref[0:1, c0:c0 + _BLK] = alive
        ident = jnp.where(pio == cio, 1.0, 0.0).astype(f32)
        alive_col = jnp.sum(ident * alive, axis=1, keepdims=True)
        mcol_ref[c0:c0 + _BLK, :] = sb * alive_col

    def out_body(cb, run):
        kb = keep_ref[0:1, pl.ds(cb * _BLK, _BLK)]
        cum = jnp.dot(kb, ut, preferred_element_type=f32) + run
        kr = jnp.where(kb > 0.5, cum - 1.0, -5.0).astype(jnp.int32)
        p3 = jnp.where(kr == oio, 1.0, 0.0).astype(f32)
        src = scols_ref[pl.ds(cb * _BLK, _BLK), :]
        out_ref[0] = out_ref[0] + jnp.dot(
            p3, src, preferred_element_type=f32,
            precision=jax.lax.Precision.HIGHEST)
        return run + jnp.sum(kb)

    out_ref[...] = jnp.zeros_like(out_ref)
    jax.lax.fori_loop(0, _NBLK, out_body, f32(0.0))


def _proposal_call(rows_all, interpret=False):
    f32 = jnp.float32
    return pl.pallas_call(
        _body,
        grid=(rows_all.shape[0],),
        in_specs=[
            pl.BlockSpec((1, 16, _NP), lambda b: (b, 0, 0)),
        ],
        out_specs=pl.BlockSpec((1, _OUTPAD, 4), lambda b: (b, 0, 0)),
        out_shape=jax.ShapeDtypeStruct((rows_all.shape[0], _OUTPAD, 4), f32),
        scratch_shapes=[
            pltpu.VMEM((8, _NPAD + _OUTPAD), f32),
            pltpu.VMEM((_NPAD, 4), f32),
            pltpu.VMEM((8, _NPAD), f32),
            pltpu.VMEM((1, _NPAD), f32),
            pltpu.VMEM((_NPAD, 1), f32),
            pltpu.VMEM((_NPAD, 1), f32),
            pltpu.VMEM((_NPAD, 4), f32),
            pltpu.VMEM((1, _NPAD), f32),
        ],
        interpret=interpret,
    )(rows_all)


def kernel(classes, bboxes, anchors):
    b = classes.shape[0]
    scores = classes[:, :, 1]
    sp = jnp.pad(scores, ((0, 0), (0, _NP - _N)), constant_values=-1.0)
    ap = jnp.pad(anchors, ((0, 0), (0, _NP - _N), (0, 0)))
    bp = jnp.pad(bboxes, ((0, 0), (0, _NP - _N), (0, 0)))
    cols_all = jnp.concatenate(
        [sp[..., None], ap, bp, jnp.zeros((b, _NP, 7), jnp.float32)], axis=2)
    rows_all = cols_all.transpose(0, 2, 1)
    out = _proposal_call(rows_all)
    return out[:, :_COUNT, :]
